# baseline (speedup 1.0000x reference)
import functools

import jax
import jax.numpy as jnp
from jax import lax
from jax.experimental import pallas as pl
from jax.experimental.pallas import tpu as pltpu
from jax.experimental.pallas import tpu_sc as plsc

NC = 2
NS = 16
NW = NC * NS
LANES = 16
CH = 128


def _body(idx_hbm, z_hbm, mean_hbm, logvar_hbm, v_hbm, mu_hbm, lv_hbm,
          idx_v, mu_v, lv_v, z_v, t_v, t_mu, t_lv,
          sem_mu, sem_lv, sem_z, sem_o):
    D = mean_hbm.shape[1]
    B = z_hbm.shape[0]
    n_chunks = idx_v.shape[0]
    wid = lax.axis_index("s") * NC + lax.axis_index("c")
    row0 = wid * n_chunks
    pltpu.sync_copy(idx_hbm.at[pl.ds(row0, n_chunks)], idx_v)
    iot = lax.broadcasted_iota(jnp.int32, (LANES,), 0)

    for c in range(n_chunks):
        base = (row0 + c) * CH
        g_mu = pltpu.async_copy(mean_hbm.at[idx_v.at[c]], mu_v, sem_mu)
        g_lv = pltpu.async_copy(logvar_hbm.at[idx_v.at[c]], lv_v, sem_lv)
        g_z = pltpu.async_copy(z_hbm.at[pl.ds(base, CH)], z_v, sem_z)
        g_mu.wait()
        g_lv.wait()
        g_z.wait()

        def row_body(r, carry):
            for k in range(D // LANES):
                sl = pl.ds(k * LANES, LANES)
                z_v[r, sl] = mu_v[r, sl] + jnp.exp(lv_v[r, sl] * 0.5) * z_v[r, sl]
            return carry

        lax.fori_loop(0, CH, row_body, 0)

        def tr_body(f, carry):
            f_idx = jnp.broadcast_to(f, (LANES,))
            for j in range(CH // LANES):
                e_idx = iot + j * LANES
                sl = pl.ds(j * LANES, LANES)
                t_v[f, sl] = plsc.load_gather(z_v, [e_idx, f_idx])
                t_mu[f, sl] = plsc.load_gather(mu_v, [e_idx, f_idx])
                t_lv[f, sl] = plsc.load_gather(lv_v, [e_idx, f_idx])
            return carry

        lax.fori_loop(0, D, tr_body, 0)

        def wr_body(f, carry):
            osl = pl.ds(f * B + base, CH)
            pltpu.async_copy(t_v.at[f], v_hbm.at[osl], sem_o)
            pltpu.async_copy(t_mu.at[f], mu_hbm.at[osl], sem_o)
            pltpu.async_copy(t_lv.at[f], lv_hbm.at[osl], sem_o)
            return carry

        lax.fori_loop(0, D, wr_body, 0)

        def wr_drain(f, carry):
            sl0 = pl.ds(0, CH)
            pltpu.make_async_copy(t_v.at[0], v_hbm.at[sl0], sem_o).wait()
            pltpu.make_async_copy(t_mu.at[0], mu_hbm.at[sl0], sem_o).wait()
            pltpu.make_async_copy(t_lv.at[0], lv_hbm.at[sl0], sem_o).wait()
            return carry

        lax.fori_loop(0, D, wr_drain, 0)


@jax.jit
def _sc_embed(i2, z2, mean2, logvar2):
    B, D = z2.shape
    n_chunks = B // (NW * CH)
    run = functools.partial(
        pl.kernel,
        out_type=[jax.ShapeDtypeStruct((D * B,), jnp.float32)] * 3,
        mesh=plsc.VectorSubcoreMesh(core_axis_name="c", subcore_axis_name="s"),
        scratch_types=[
            pltpu.VMEM((n_chunks, CH), jnp.int32),
            pltpu.VMEM((CH, D), jnp.float32),
            pltpu.VMEM((CH, D), jnp.float32),
            pltpu.VMEM((CH, D), jnp.float32),
            pltpu.VMEM((D, CH), jnp.float32),
            pltpu.VMEM((D, CH), jnp.float32),
            pltpu.VMEM((D, CH), jnp.float32),
            pltpu.SemaphoreType.DMA,
            pltpu.SemaphoreType.DMA,
            pltpu.SemaphoreType.DMA,
            pltpu.SemaphoreType.DMA,
        ],
        compiler_params=pltpu.CompilerParams(
            use_tc_tiling_on_sc=False, needs_layout_passes=False),
    )(_body)
    return run(i2, z2, mean2, logvar2)


def kernel(i, z, mean, logvar):
    B, W, L = z.shape
    N = mean.shape[0]
    D = W * L
    v1, mu1, lv1 = _sc_embed(
        i.astype(jnp.int32).reshape(B // CH, CH),
        z.reshape(B, D),
        mean.reshape(N, D),
        logvar.reshape(N, D),
    )

    def back(t):
        return t.reshape(W, L, B).transpose(2, 0, 1)

    return (back(v1), back(mu1), back(lv1))
